# baseline (speedup 1.0000x reference)
import functools

import jax
import jax.numpy as jnp
from jax import lax
from jax.experimental import pallas as pl
from jax.experimental.pallas import tpu as pltpu
from jax.experimental.pallas import tpu_sc as plsc

N = 10000
IN = 128
H = 64
C = 32

NPAD = 10240
CH = 128
NC = 2
NS = 16
NW = NC * NS
ROWS_PER_TILE = NPAD // NS
NB = 8


def _make_sc_agg(nchunks: int, with_deg: bool):
    out_type = [jax.ShapeDtypeStruct((NC, NPAD, H), jnp.float32)]
    if with_deg:
        out_type.append(jax.ShapeDtypeStruct((NC, NPAD), jnp.float32))

    scratch = [
        pltpu.VMEM((nchunks, CH), jnp.int32),
        pltpu.VMEM((nchunks, CH), jnp.int32),
        pltpu.VMEM((NB, CH, H), jnp.float32),
        pltpu.VMEM((CH,), jnp.float32),
        pltpu.VMEM_SHARED((NPAD, H), jnp.float32),
        pltpu.VMEM_SHARED((NPAD,), jnp.float32),
    ] + [pltpu.SemaphoreType.DMA] * NB

    mesh = plsc.VectorSubcoreMesh(core_axis_name="c", subcore_axis_name="s",
                                  num_cores=NC, num_subcores=NS)

    def body(table, src, dst, z64, z1, *rest):
        if with_deg:
            out, degout, sidx, didx, rows, ones, acc, dacc = rest[:8]
        else:
            out, sidx, didx, rows, ones, acc, dacc = rest[:7]
            degout = None
        sems = rest[-NB:]
        c = lax.axis_index("c")
        s = lax.axis_index("s")
        tid = c * NS + s

        for i in range(CH // 16):
            ones[pl.ds(i * 16, 16)] = jnp.ones((16,), jnp.float32)

        @pl.when(s == 0)
        def _():
            pltpu.sync_copy(z64, acc)
            pltpu.sync_copy(z1, dacc)

        pltpu.sync_copy(src.at[tid], sidx)
        pltpu.sync_copy(dst.at[tid], didx)
        plsc.subcore_barrier()

        for b in range(NB):
            pltpu.async_copy(table.at[sidx.at[b]], rows.at[b], sems[b])

        def group(g, carry):
            for b in range(NB):
                j = g * NB + b
                pltpu.make_async_copy(table.at[sidx.at[j]], rows.at[b],
                                      sems[b]).wait()
                pltpu.sync_copy(rows.at[b], acc.at[didx.at[j]], add=True)
                if with_deg:
                    pltpu.sync_copy(ones, dacc.at[didx.at[j]], add=True)

                @pl.when(j + NB < nchunks)
                def _():
                    pltpu.async_copy(table.at[sidx.at[j + NB]], rows.at[b],
                                     sems[b])
            return carry

        lax.fori_loop(0, nchunks // NB, group, 0)
        plsc.subcore_barrier()

        r0 = s * ROWS_PER_TILE
        pltpu.sync_copy(acc.at[pl.ds(r0, ROWS_PER_TILE)],
                        out.at[c, pl.ds(r0, ROWS_PER_TILE)])
        if with_deg:
            pltpu.sync_copy(dacc.at[pl.ds(r0, ROWS_PER_TILE)],
                            degout.at[c, pl.ds(r0, ROWS_PER_TILE)])

    return pl.kernel(body, out_type=out_type, mesh=mesh, scratch_types=scratch,
                     compiler_params=pltpu.CompilerParams(use_tc_tiling_on_sc=False))


BN = 2000


def _tc1_body(x_ref, wl_ref, wr_ref, y_ref, r_ref):
    x = x_ref[...]
    y_ref[...] = jnp.dot(x, wl_ref[...], preferred_element_type=jnp.float32)
    r_ref[...] = jnp.dot(x, wr_ref[...], preferred_element_type=jnp.float32)


def _tc2_body(p_ref, dpt_ref, r1_ref, b1_ref, wl_ref, wr_ref, y_ref, r_ref):
    agg = p_ref[0] + p_ref[1]
    deg = dpt_ref[:, 0] + dpt_ref[:, 1]
    inv = 1.0 / jnp.maximum(deg, 1.0)
    h = jnp.maximum(agg * inv[:, None] + r1_ref[...] + b1_ref[...][None, :], 0.0)
    y_ref[...] = jnp.dot(h, wl_ref[...], preferred_element_type=jnp.float32)
    r_ref[...] = jnp.dot(h, wr_ref[...], preferred_element_type=jnp.float32)


def _tc3_body(p_ref, dpt_ref, r2_ref, b2_ref, wc_ref, bc_ref, log_ref, emb_ref):
    agg = p_ref[0] + p_ref[1]
    deg = dpt_ref[:, 0] + dpt_ref[:, 1]
    inv = 1.0 / jnp.maximum(deg, 1.0)
    emb = agg * inv[:, None] + r2_ref[...] + b2_ref[...][None, :]
    emb_ref[...] = emb
    log_ref[...] = (jnp.dot(emb, wc_ref[...], preferred_element_type=jnp.float32)
                    + bc_ref[...][None, :])


def _row_spec(d):
    return pl.BlockSpec((BN, d), lambda i: (i, 0))


def _full_spec(shape):
    return pl.BlockSpec(shape, lambda i: tuple(0 for _ in shape))


_tc1 = pl.pallas_call(
    _tc1_body,
    grid=(N // BN,),
    in_specs=[_row_spec(IN), _full_spec((IN, H)), _full_spec((IN, H))],
    out_specs=[_row_spec(H), _row_spec(H)],
    out_shape=[jax.ShapeDtypeStruct((N, H), jnp.float32)] * 2,
)

_tc2 = pl.pallas_call(
    _tc2_body,
    grid=(N // BN,),
    in_specs=[
        pl.BlockSpec((NC, BN, H), lambda i: (0, i, 0)),
        _row_spec(NC),
        _row_spec(H),
        _full_spec((H,)),
        _full_spec((H, H)),
        _full_spec((H, H)),
    ],
    out_specs=[_row_spec(H), _row_spec(H)],
    out_shape=[jax.ShapeDtypeStruct((N, H), jnp.float32)] * 2,
)

_tc3 = pl.pallas_call(
    _tc3_body,
    grid=(N // BN,),
    in_specs=[
        pl.BlockSpec((NC, BN, H), lambda i: (0, i, 0)),
        _row_spec(NC),
        _row_spec(H),
        _full_spec((H,)),
        _full_spec((H, C)),
        _full_spec((C,)),
    ],
    out_specs=[_row_spec(C), _row_spec(H)],
    out_shape=[jax.ShapeDtypeStruct((N, C), jnp.float32),
               jax.ShapeDtypeStruct((N, H), jnp.float32)],
)


def kernel(x, edge_index, W1l, b1, W1r, W2l, b2, W2r, Wc, bc):
    E = edge_index.shape[1]
    nchunks = -(-(-(-E // (NW * CH))) // NB) * NB
    e_pad = NW * nchunks * CH

    src = edge_index[0]
    dst = edge_index[1]
    pad = e_pad - E
    if pad:
        src = jnp.concatenate([src, jnp.zeros((pad,), jnp.int32)])
        dst = jnp.concatenate([dst, jnp.full((pad,), NPAD - 1, jnp.int32)])
    src = src.reshape(NW, nchunks, CH)
    dst = dst.reshape(NW, nchunks, CH)

    z64 = jnp.zeros((NPAD, H), jnp.float32)
    z1 = jnp.zeros((NPAD,), jnp.float32)

    sc_agg1 = _make_sc_agg(nchunks, with_deg=True)
    sc_agg2 = _make_sc_agg(nchunks, with_deg=False)

    y1, r1 = _tc1(x, W1l, W1r)
    p1, dp = sc_agg1(y1, src, dst, z64, z1)
    dpt = dp.T
    y2, r2 = _tc2(p1, dpt, r1, b1, W2l, W2r)
    p2 = sc_agg2(y2, src, dst, z64, z1)
    p2 = p2[0] if isinstance(p2, (list, tuple)) else p2
    logits, emb = _tc3(p2, dpt, r2, b2, Wc, bc)
    return logits, emb


# speedup vs baseline: 2.1525x; 2.1525x over previous
import functools

import jax
import jax.numpy as jnp
from jax import lax
from jax.experimental import pallas as pl
from jax.experimental.pallas import tpu as pltpu
from jax.experimental.pallas import tpu_sc as plsc

N = 10000
IN = 128
H = 64
C = 32

NPAD = 10240
CH = 128
NC = 2
NS = 16
NW = NC * NS
ROWS_PER_TILE = NPAD // NS
NB = 8


HH = H // NC


def _make_sc_agg(nchunks: int, with_deg: bool):
    out_type = [jax.ShapeDtypeStruct((NC, NPAD, HH), jnp.float32)]
    if with_deg:
        out_type.append(jax.ShapeDtypeStruct((NC, NPAD), jnp.float32))

    scratch = [
        pltpu.VMEM((nchunks, CH), jnp.int32),
        pltpu.VMEM((nchunks, CH), jnp.int32),
        pltpu.VMEM((NB, CH, HH), jnp.float32),
        pltpu.VMEM((CH,), jnp.float32),
        pltpu.VMEM_SHARED((NPAD, HH), jnp.float32),
        pltpu.VMEM_SHARED((NPAD,), jnp.float32),
        pltpu.VMEM_SHARED((N, HH), jnp.float32),
    ] + [pltpu.SemaphoreType.DMA] * NB

    mesh = plsc.VectorSubcoreMesh(core_axis_name="c", subcore_axis_name="s",
                                  num_cores=NC, num_subcores=NS)
    nch2 = nchunks // 2

    def body(table, src, dst, zh, z1, *rest):
        if with_deg:
            out, degout, sidx, didx, rows, ones, acc, dacc, tab = rest[:9]
        else:
            out, sidx, didx, rows, ones, acc, dacc, tab = rest[:8]
            degout = None
        sems = rest[-NB:]
        c = lax.axis_index("c")
        s = lax.axis_index("s")

        for i in range(CH // 16):
            ones[pl.ds(i * 16, 16)] = jnp.ones((16,), jnp.float32)

        @pl.when(s == 0)
        def _():
            pltpu.sync_copy(zh, acc)
            pltpu.sync_copy(z1, dacc)
            pltpu.sync_copy(table.at[c], tab)

        pltpu.sync_copy(src.at[s], sidx)
        pltpu.sync_copy(dst.at[s], didx)
        plsc.subcore_barrier()

        for b in range(NB):
            pltpu.async_copy(tab.at[sidx.at[b]], rows.at[b], sems[b])

        def group(g, carry):
            for b in range(NB):
                j = g * NB + b
                pltpu.make_async_copy(tab.at[sidx.at[j]], rows.at[b],
                                      sems[b]).wait()
                pltpu.sync_copy(rows.at[b], acc.at[didx.at[j]], add=True)
                if with_deg:
                    @pl.when((j < nch2) == (c == 0))
                    def _():
                        pltpu.sync_copy(ones, dacc.at[didx.at[j]], add=True)

                @pl.when(j + NB < nchunks)
                def _():
                    pltpu.async_copy(tab.at[sidx.at[j + NB]], rows.at[b],
                                     sems[b])
            return carry

        lax.fori_loop(0, nchunks // NB, group, 0)
        plsc.subcore_barrier()

        r0 = s * ROWS_PER_TILE
        pltpu.sync_copy(acc.at[pl.ds(r0, ROWS_PER_TILE)],
                        out.at[c, pl.ds(r0, ROWS_PER_TILE)])
        if with_deg:
            pltpu.sync_copy(dacc.at[pl.ds(r0, ROWS_PER_TILE)],
                            degout.at[c, pl.ds(r0, ROWS_PER_TILE)])

    return pl.kernel(body, out_type=out_type, mesh=mesh, scratch_types=scratch,
                     compiler_params=pltpu.CompilerParams(use_tc_tiling_on_sc=False))


BN = 2000


def _tc1_body(x_ref, wl_ref, wr_ref, y_ref, r_ref):
    x = x_ref[...]
    y = jnp.dot(x, wl_ref[...], preferred_element_type=jnp.float32)
    y_ref[0] = y[:, :HH]
    y_ref[1] = y[:, HH:]
    r_ref[...] = jnp.dot(x, wr_ref[...], preferred_element_type=jnp.float32)


def _tc2_body(p_ref, dpt_ref, r1_ref, b1_ref, wl_ref, wr_ref, y_ref, r_ref):
    agg = jnp.concatenate([p_ref[0], p_ref[1]], axis=-1)
    deg = dpt_ref[:, 0] + dpt_ref[:, 1]
    inv = 1.0 / jnp.maximum(deg, 1.0)
    h = jnp.maximum(agg * inv[:, None] + r1_ref[...] + b1_ref[...][None, :], 0.0)
    y = jnp.dot(h, wl_ref[...], preferred_element_type=jnp.float32)
    y_ref[0] = y[:, :HH]
    y_ref[1] = y[:, HH:]
    r_ref[...] = jnp.dot(h, wr_ref[...], preferred_element_type=jnp.float32)


def _tc3_body(p_ref, dpt_ref, r2_ref, b2_ref, wc_ref, bc_ref, log_ref, emb_ref):
    agg = jnp.concatenate([p_ref[0], p_ref[1]], axis=-1)
    deg = dpt_ref[:, 0] + dpt_ref[:, 1]
    inv = 1.0 / jnp.maximum(deg, 1.0)
    emb = agg * inv[:, None] + r2_ref[...] + b2_ref[...][None, :]
    emb_ref[...] = emb
    log_ref[...] = (jnp.dot(emb, wc_ref[...], preferred_element_type=jnp.float32)
                    + bc_ref[...][None, :])


def _row_spec(d):
    return pl.BlockSpec((BN, d), lambda i: (i, 0))


def _full_spec(shape):
    return pl.BlockSpec(shape, lambda i: tuple(0 for _ in shape))


_split_spec = pl.BlockSpec((NC, BN, HH), lambda i: (0, i, 0))

_tc1 = pl.pallas_call(
    _tc1_body,
    grid=(N // BN,),
    in_specs=[_row_spec(IN), _full_spec((IN, H)), _full_spec((IN, H))],
    out_specs=[_split_spec, _row_spec(H)],
    out_shape=[jax.ShapeDtypeStruct((NC, N, HH), jnp.float32),
               jax.ShapeDtypeStruct((N, H), jnp.float32)],
)

_tc2 = pl.pallas_call(
    _tc2_body,
    grid=(N // BN,),
    in_specs=[
        _split_spec,
        _row_spec(NC),
        _row_spec(H),
        _full_spec((H,)),
        _full_spec((H, H)),
        _full_spec((H, H)),
    ],
    out_specs=[_split_spec, _row_spec(H)],
    out_shape=[jax.ShapeDtypeStruct((NC, N, HH), jnp.float32),
               jax.ShapeDtypeStruct((N, H), jnp.float32)],
)

_tc3 = pl.pallas_call(
    _tc3_body,
    grid=(N // BN,),
    in_specs=[
        _split_spec,
        _row_spec(NC),
        _row_spec(H),
        _full_spec((H,)),
        _full_spec((H, C)),
        _full_spec((C,)),
    ],
    out_specs=[_row_spec(C), _row_spec(H)],
    out_shape=[jax.ShapeDtypeStruct((N, C), jnp.float32),
               jax.ShapeDtypeStruct((N, H), jnp.float32)],
)


def kernel(x, edge_index, W1l, b1, W1r, W2l, b2, W2r, Wc, bc):
    E = edge_index.shape[1]
    nchunks = -(-(-(-E // (NS * CH))) // NB) * NB
    e_pad = NS * nchunks * CH

    src = edge_index[0]
    dst = edge_index[1]
    pad = e_pad - E
    if pad:
        src = jnp.concatenate([src, jnp.zeros((pad,), jnp.int32)])
        dst = jnp.concatenate([dst, jnp.full((pad,), NPAD - 1, jnp.int32)])
    src = src.reshape(NS, nchunks, CH)
    dst = dst.reshape(NS, nchunks, CH)

    zh = jnp.zeros((NPAD, HH), jnp.float32)
    z1 = jnp.zeros((NPAD,), jnp.float32)

    sc_agg1 = _make_sc_agg(nchunks, with_deg=True)
    sc_agg2 = _make_sc_agg(nchunks, with_deg=False)

    y1, r1 = _tc1(x, W1l, W1r)
    p1, dp = sc_agg1(y1, src, dst, zh, z1)
    dpt = dp.T
    y2, r2 = _tc2(p1, dpt, r1, b1, W2l, W2r)
    p2 = sc_agg2(y2, src, dst, zh, z1)
    p2 = p2[0] if isinstance(p2, (list, tuple)) else p2
    logits, emb = _tc3(p2, dpt, r2, b2, Wc, bc)
    return logits, emb


# speedup vs baseline: 2.2636x; 1.0516x over previous
import functools

import jax
import jax.numpy as jnp
from jax import lax
from jax.experimental import pallas as pl
from jax.experimental.pallas import tpu as pltpu
from jax.experimental.pallas import tpu_sc as plsc

N = 10000
IN = 128
H = 64
C = 32

NPAD = 10240
CH = 128
NC = 2
NS = 16
NW = NC * NS
ROWS_PER_TILE = NPAD // NS
NB = 4
NPOOL = 2 * NB


HH = H // NC


def _make_sc_agg(nchunks: int, with_deg: bool):
    out_type = [jax.ShapeDtypeStruct((NC, NPAD, HH), jnp.float32)]
    if with_deg:
        out_type.append(jax.ShapeDtypeStruct((NC, NPAD), jnp.float32))

    scratch = [
        pltpu.VMEM((nchunks * CH,), jnp.int32),
        pltpu.VMEM((nchunks * CH,), jnp.int32),
        pltpu.VMEM((NPOOL, CH, HH), jnp.float32),
        pltpu.VMEM((CH,), jnp.float32),
        pltpu.VMEM_SHARED((NPAD, HH), jnp.float32),
        pltpu.VMEM_SHARED((NPAD,), jnp.float32),
        pltpu.VMEM_SHARED((N, HH), jnp.float32),
    ] + [pltpu.SemaphoreType.DMA] * (2 * NPOOL)

    mesh = plsc.VectorSubcoreMesh(core_axis_name="c", subcore_axis_name="s",
                                  num_cores=NC, num_subcores=NS)
    nch2 = nchunks // 2

    def body(table, src, dst, zh, z1, *rest):
        if with_deg:
            out, degout, sidx, didx, rows, ones, acc, dacc, tab = rest[:9]
        else:
            out, sidx, didx, rows, ones, acc, dacc, tab = rest[:8]
            degout = None
        sems = rest[-2 * NPOOL:]
        c = lax.axis_index("c")
        s = lax.axis_index("s")

        for i in range(CH // 16):
            ones[pl.ds(i * 16, 16)] = jnp.ones((16,), jnp.float32)

        @pl.when(s == 0)
        def _():
            pltpu.sync_copy(zh, acc)
            pltpu.sync_copy(z1, dacc)
            pltpu.sync_copy(table.at[c], tab)

        epw = nchunks * CH
        pltpu.sync_copy(src.at[pl.ds(s * epw, epw)], sidx)
        pltpu.sync_copy(dst.at[pl.ds(s * epw, epw)], didx)
        plsc.subcore_barrier()

        def sidx_of(j):
            return sidx.at[pl.ds(j * CH, CH)]

        def didx_of(j):
            return didx.at[pl.ds(j * CH, CH)]

        for b in range(NB):
            pltpu.async_copy(tab.at[sidx_of(b)], rows.at[b], sems[b])

        def pair(p, carry):
            for u in range(NPOOL):
                j = p * NPOOL + u
                b = u
                bn = (u + NB) % NPOOL
                pltpu.make_async_copy(tab.at[sidx_of(j)], rows.at[b],
                                      sems[b]).wait()
                pltpu.async_copy(rows.at[b], acc.at[didx_of(j)],
                                 sems[NPOOL + b], add=True)
                if with_deg:
                    @pl.when((j < nch2) == (c == 0))
                    def _():
                        pltpu.sync_copy(ones, dacc.at[didx_of(j)], add=True)

                @pl.when(jnp.logical_and(j + NB < nchunks, j >= NB))
                def _():
                    pltpu.make_async_copy(rows.at[bn], acc.at[didx_of(0)],
                                          sems[NPOOL + bn]).wait()

                @pl.when(j + NB < nchunks)
                def _():
                    pltpu.async_copy(tab.at[sidx_of(j + NB)], rows.at[bn],
                                     sems[bn])
            return carry

        lax.fori_loop(0, nchunks // NPOOL, pair, 0)

        for b in range(NPOOL):
            pltpu.make_async_copy(rows.at[b], acc.at[didx_of(0)],
                                  sems[NPOOL + b]).wait()
        plsc.subcore_barrier()

        r0 = s * ROWS_PER_TILE
        pltpu.sync_copy(acc.at[pl.ds(r0, ROWS_PER_TILE)],
                        out.at[c, pl.ds(r0, ROWS_PER_TILE)])
        if with_deg:
            pltpu.sync_copy(dacc.at[pl.ds(r0, ROWS_PER_TILE)],
                            degout.at[c, pl.ds(r0, ROWS_PER_TILE)])

    return pl.kernel(body, out_type=out_type, mesh=mesh, scratch_types=scratch,
                     compiler_params=pltpu.CompilerParams(use_tc_tiling_on_sc=False))


BN = 2000


def _tc1_body(x_ref, wl_ref, wr_ref, y_ref, r_ref):
    x = x_ref[...]
    y = jnp.dot(x, wl_ref[...], preferred_element_type=jnp.float32)
    y_ref[0] = y[:, :HH]
    y_ref[1] = y[:, HH:]
    r_ref[...] = jnp.dot(x, wr_ref[...], preferred_element_type=jnp.float32)


def _tc2_body(p_ref, dpt_ref, r1_ref, b1_ref, wl_ref, wr_ref, y_ref, r_ref):
    agg = jnp.concatenate([p_ref[0], p_ref[1]], axis=-1)
    deg = dpt_ref[:, 0] + dpt_ref[:, 1]
    inv = 1.0 / jnp.maximum(deg, 1.0)
    h = jnp.maximum(agg * inv[:, None] + r1_ref[...] + b1_ref[...][None, :], 0.0)
    y = jnp.dot(h, wl_ref[...], preferred_element_type=jnp.float32)
    y_ref[0] = y[:, :HH]
    y_ref[1] = y[:, HH:]
    r_ref[...] = jnp.dot(h, wr_ref[...], preferred_element_type=jnp.float32)


def _tc3_body(p_ref, dpt_ref, r2_ref, b2_ref, wc_ref, bc_ref, log_ref, emb_ref):
    agg = jnp.concatenate([p_ref[0], p_ref[1]], axis=-1)
    deg = dpt_ref[:, 0] + dpt_ref[:, 1]
    inv = 1.0 / jnp.maximum(deg, 1.0)
    emb = agg * inv[:, None] + r2_ref[...] + b2_ref[...][None, :]
    emb_ref[...] = emb
    log_ref[...] = (jnp.dot(emb, wc_ref[...], preferred_element_type=jnp.float32)
                    + bc_ref[...][None, :])


def _row_spec(d):
    return pl.BlockSpec((BN, d), lambda i: (i, 0))


def _full_spec(shape):
    return pl.BlockSpec(shape, lambda i: tuple(0 for _ in shape))


_split_spec = pl.BlockSpec((NC, BN, HH), lambda i: (0, i, 0))

_tc1 = pl.pallas_call(
    _tc1_body,
    grid=(N // BN,),
    in_specs=[_row_spec(IN), _full_spec((IN, H)), _full_spec((IN, H))],
    out_specs=[_split_spec, _row_spec(H)],
    out_shape=[jax.ShapeDtypeStruct((NC, N, HH), jnp.float32),
               jax.ShapeDtypeStruct((N, H), jnp.float32)],
)

_tc2 = pl.pallas_call(
    _tc2_body,
    grid=(N // BN,),
    in_specs=[
        _split_spec,
        _row_spec(NC),
        _row_spec(H),
        _full_spec((H,)),
        _full_spec((H, H)),
        _full_spec((H, H)),
    ],
    out_specs=[_split_spec, _row_spec(H)],
    out_shape=[jax.ShapeDtypeStruct((NC, N, HH), jnp.float32),
               jax.ShapeDtypeStruct((N, H), jnp.float32)],
)

_tc3 = pl.pallas_call(
    _tc3_body,
    grid=(N // BN,),
    in_specs=[
        _split_spec,
        _row_spec(NC),
        _row_spec(H),
        _full_spec((H,)),
        _full_spec((H, C)),
        _full_spec((C,)),
    ],
    out_specs=[_row_spec(C), _row_spec(H)],
    out_shape=[jax.ShapeDtypeStruct((N, C), jnp.float32),
               jax.ShapeDtypeStruct((N, H), jnp.float32)],
)


def kernel(x, edge_index, W1l, b1, W1r, W2l, b2, W2r, Wc, bc):
    E = edge_index.shape[1]
    nchunks = -(-(-(-E // (NS * CH))) // NPOOL) * NPOOL
    e_pad = NS * nchunks * CH

    src = edge_index[0]
    dst = edge_index[1]
    pad = e_pad - E
    if pad:
        src = jnp.concatenate([src, jnp.zeros((pad,), jnp.int32)])
        dst = jnp.concatenate([dst, jnp.full((pad,), NPAD - 1, jnp.int32)])

    zh = jnp.zeros((NPAD, HH), jnp.float32)
    z1 = jnp.zeros((NPAD,), jnp.float32)

    sc_agg1 = _make_sc_agg(nchunks, with_deg=True)
    sc_agg2 = _make_sc_agg(nchunks, with_deg=False)

    y1, r1 = _tc1(x, W1l, W1r)
    p1, dp = sc_agg1(y1, src, dst, zh, z1)
    dpt = dp.T
    y2, r2 = _tc2(p1, dpt, r1, b1, W2l, W2r)
    p2 = sc_agg2(y2, src, dst, zh, z1)
    p2 = p2[0] if isinstance(p2, (list, tuple)) else p2
    logits, emb = _tc3(p2, dpt, r2, b2, Wc, bc)
    return logits, emb


# speedup vs baseline: 2.3345x; 1.0313x over previous
import functools

import jax
import jax.numpy as jnp
from jax import lax
from jax.experimental import pallas as pl
from jax.experimental.pallas import tpu as pltpu
from jax.experimental.pallas import tpu_sc as plsc

N = 10000
IN = 128
H = 64
C = 32

NPAD = 10240
CH = 128
NC = 2
NS = 16
NW = NC * NS
ROWS_PER_TILE = NPAD // NS
NB = 4
NPOOL = 2 * NB


HH = H // NC


def _make_sc_agg(nchunks: int, with_deg: bool):
    out_type = [jax.ShapeDtypeStruct((NC, NPAD, HH), jnp.float32)]
    if with_deg:
        out_type.append(jax.ShapeDtypeStruct((NC, NPAD), jnp.float32))

    scratch = [
        pltpu.VMEM((nchunks, CH), jnp.int32),
        pltpu.VMEM((nchunks, CH), jnp.int32),
        pltpu.VMEM((NPOOL, CH, HH), jnp.float32),
        pltpu.VMEM((CH,), jnp.float32),
        pltpu.VMEM_SHARED((NPAD, HH), jnp.float32),
        pltpu.VMEM_SHARED((NPAD,), jnp.float32),
        pltpu.VMEM_SHARED((N, HH), jnp.float32),
    ] + [pltpu.SemaphoreType.DMA] * (2 * NPOOL)

    mesh = plsc.VectorSubcoreMesh(core_axis_name="c", subcore_axis_name="s",
                                  num_cores=NC, num_subcores=NS)
    nch2 = nchunks // 2

    def body(table, src, dst, zh, z1, *rest):
        if with_deg:
            out, degout, sidx, didx, rows, ones, acc, dacc, tab = rest[:9]
        else:
            out, sidx, didx, rows, ones, acc, dacc, tab = rest[:8]
            degout = None
        sems = rest[-2 * NPOOL:]
        c = lax.axis_index("c")
        s = lax.axis_index("s")

        for i in range(CH // 16):
            ones[pl.ds(i * 16, 16)] = jnp.ones((16,), jnp.float32)

        @pl.when(s == 0)
        def _():
            pltpu.sync_copy(zh, acc)
            pltpu.sync_copy(z1, dacc)
            pltpu.sync_copy(table.at[c], tab)

        pltpu.sync_copy(src.at[pl.ds(s * nchunks, nchunks)], sidx)
        pltpu.sync_copy(dst.at[pl.ds(s * nchunks, nchunks)], didx)
        plsc.subcore_barrier()

        def sidx_of(j):
            return sidx.at[j]

        def didx_of(j):
            return didx.at[j]

        for b in range(NB):
            pltpu.async_copy(tab.at[sidx_of(b)], rows.at[b], sems[b])

        def pair(p, carry):
            for u in range(NPOOL):
                j = p * NPOOL + u
                b = u
                bn = (u + NB) % NPOOL
                pltpu.make_async_copy(tab.at[sidx_of(j)], rows.at[b],
                                      sems[b]).wait()
                pltpu.async_copy(rows.at[b], acc.at[didx_of(j)],
                                 sems[NPOOL + b], add=True)
                if with_deg:
                    @pl.when((j < nch2) == (c == 0))
                    def _():
                        pltpu.sync_copy(ones, dacc.at[didx_of(j)], add=True)

                @pl.when(jnp.logical_and(j + NB < nchunks, j >= NB))
                def _():
                    pltpu.make_async_copy(rows.at[bn], acc.at[didx_of(0)],
                                          sems[NPOOL + bn]).wait()

                @pl.when(j + NB < nchunks)
                def _():
                    pltpu.async_copy(tab.at[sidx_of(j + NB)], rows.at[bn],
                                     sems[bn])
            return carry

        lax.fori_loop(0, nchunks // NPOOL, pair, 0)

        for b in range(NPOOL):
            pltpu.make_async_copy(rows.at[b], acc.at[didx_of(0)],
                                  sems[NPOOL + b]).wait()
        plsc.subcore_barrier()

        r0 = s * ROWS_PER_TILE
        pltpu.sync_copy(acc.at[pl.ds(r0, ROWS_PER_TILE)],
                        out.at[c, pl.ds(r0, ROWS_PER_TILE)])
        if with_deg:
            pltpu.sync_copy(dacc.at[pl.ds(r0, ROWS_PER_TILE)],
                            degout.at[c, pl.ds(r0, ROWS_PER_TILE)])

    return pl.kernel(body, out_type=out_type, mesh=mesh, scratch_types=scratch,
                     compiler_params=pltpu.CompilerParams(use_tc_tiling_on_sc=False))


def _tc1_body(x_ref, wl_ref, wr_ref, y_ref, r_ref):
    x = x_ref[...]
    y = jnp.dot(x, wl_ref[...], preferred_element_type=jnp.float32)
    y_ref[0] = y[:, :HH]
    y_ref[1] = y[:, HH:]
    r_ref[...] = jnp.dot(x, wr_ref[...], preferred_element_type=jnp.float32)


def _tc2_body(p_ref, dpt_ref, r1_ref, b1_ref, wl_ref, wr_ref, y_ref, r_ref):
    agg = jnp.concatenate([p_ref[0, :N], p_ref[1, :N]], axis=-1)
    deg = dpt_ref[:N, 0] + dpt_ref[:N, 1]
    inv = 1.0 / jnp.maximum(deg, 1.0)
    h = jnp.maximum(agg * inv[:, None] + r1_ref[...] + b1_ref[...][None, :], 0.0)
    y = jnp.dot(h, wl_ref[...], preferred_element_type=jnp.float32)
    y_ref[0] = y[:, :HH]
    y_ref[1] = y[:, HH:]
    r_ref[...] = jnp.dot(h, wr_ref[...], preferred_element_type=jnp.float32)


def _tc3_body(p_ref, dpt_ref, r2_ref, b2_ref, wc_ref, bc_ref, log_ref, emb_ref):
    agg = jnp.concatenate([p_ref[0, :N], p_ref[1, :N]], axis=-1)
    deg = dpt_ref[:N, 0] + dpt_ref[:N, 1]
    inv = 1.0 / jnp.maximum(deg, 1.0)
    emb = agg * inv[:, None] + r2_ref[...] + b2_ref[...][None, :]
    emb_ref[...] = emb
    log_ref[...] = (jnp.dot(emb, wc_ref[...], preferred_element_type=jnp.float32)
                    + bc_ref[...][None, :])


def _whole(shape):
    return pl.BlockSpec(shape, lambda i: tuple(0 for _ in shape))


_tc1 = pl.pallas_call(
    _tc1_body,
    grid=(1,),
    in_specs=[_whole((N, IN)), _whole((IN, H)), _whole((IN, H))],
    out_specs=[_whole((NC, N, HH)), _whole((N, H))],
    out_shape=[jax.ShapeDtypeStruct((NC, N, HH), jnp.float32),
               jax.ShapeDtypeStruct((N, H), jnp.float32)],
)

_tc2 = pl.pallas_call(
    _tc2_body,
    grid=(1,),
    in_specs=[
        _whole((NC, NPAD, HH)),
        _whole((NPAD, NC)),
        _whole((N, H)),
        _whole((H,)),
        _whole((H, H)),
        _whole((H, H)),
    ],
    out_specs=[_whole((NC, N, HH)), _whole((N, H))],
    out_shape=[jax.ShapeDtypeStruct((NC, N, HH), jnp.float32),
               jax.ShapeDtypeStruct((N, H), jnp.float32)],
)

_tc3 = pl.pallas_call(
    _tc3_body,
    grid=(1,),
    in_specs=[
        _whole((NC, NPAD, HH)),
        _whole((NPAD, NC)),
        _whole((N, H)),
        _whole((H,)),
        _whole((H, C)),
        _whole((C,)),
    ],
    out_specs=[_whole((N, C)), _whole((N, H))],
    out_shape=[jax.ShapeDtypeStruct((N, C), jnp.float32),
               jax.ShapeDtypeStruct((N, H), jnp.float32)],
)


def _make_edge_prep(E: int, e_pad: int):
    nrows = e_pad // CH

    def body(ei_ref, srcm_ref, dstm_ref):
        sm = ei_ref[0].reshape(nrows, CH)
        dm = ei_ref[1].reshape(nrows, CH)
        flat = (lax.broadcasted_iota(jnp.int32, (nrows, CH), 0) * CH +
                lax.broadcasted_iota(jnp.int32, (nrows, CH), 1))
        mask = flat < E
        srcm_ref[...] = jnp.where(mask, sm, 0)
        dstm_ref[...] = jnp.where(mask, dm, NPAD - 1)

    return pl.pallas_call(
        body,
        grid=(1,),
        in_specs=[pl.BlockSpec((2, e_pad), lambda i: (0, 0))],
        out_specs=[_whole((nrows, CH)), _whole((nrows, CH))],
        out_shape=[jax.ShapeDtypeStruct((nrows, CH), jnp.int32)] * 2,
    )


def kernel(x, edge_index, W1l, b1, W1r, W2l, b2, W2r, Wc, bc):
    E = edge_index.shape[1]
    nchunks = -(-(-(-E // (NS * CH))) // NPOOL) * NPOOL
    e_pad = NS * nchunks * CH

    srcm, dstm = _make_edge_prep(E, e_pad)(edge_index)

    zh = jnp.zeros((NPAD, HH), jnp.float32)
    z1 = jnp.zeros((NPAD,), jnp.float32)

    sc_agg1 = _make_sc_agg(nchunks, with_deg=True)
    sc_agg2 = _make_sc_agg(nchunks, with_deg=False)

    y1, r1 = _tc1(x, W1l, W1r)
    p1, dp = sc_agg1(y1, srcm, dstm, zh, z1)
    dpt = dp.T
    y2, r2 = _tc2(p1, dpt, r1, b1, W2l, W2r)
    p2 = sc_agg2(y2, srcm, dstm, zh, z1)
    p2 = p2[0] if isinstance(p2, (list, tuple)) else p2
    logits, emb = _tc3(p2, dpt, r2, b2, Wc, bc)
    return logits, emb


# speedup vs baseline: 2.3454x; 1.0047x over previous
import functools

import jax
import jax.numpy as jnp
from jax import lax
from jax.experimental import pallas as pl
from jax.experimental.pallas import tpu as pltpu
from jax.experimental.pallas import tpu_sc as plsc

N = 10000
IN = 128
H = 64
C = 32

NPAD = 10240
CH = 128
NC = 2
NS = 16
NW = NC * NS
ROWS_PER_TILE = NPAD // NS
NB = 5
NPOOL = 2 * NB


HH = H // NC


def _make_sc_agg(nchunks: int, with_deg: bool):
    out_type = [jax.ShapeDtypeStruct((NC, NPAD, HH), jnp.float32)]
    if with_deg:
        out_type.append(jax.ShapeDtypeStruct((NC, NPAD), jnp.float32))

    scratch = [
        pltpu.VMEM((nchunks, CH), jnp.int32),
        pltpu.VMEM((nchunks, CH), jnp.int32),
        pltpu.VMEM((NPOOL, CH, HH), jnp.float32),
        pltpu.VMEM((CH,), jnp.float32),
        pltpu.VMEM_SHARED((NPAD, HH), jnp.float32),
        pltpu.VMEM_SHARED((NPAD,), jnp.float32),
        pltpu.VMEM_SHARED((N, HH), jnp.float32),
    ] + [pltpu.SemaphoreType.DMA] * (2 * NPOOL)

    mesh = plsc.VectorSubcoreMesh(core_axis_name="c", subcore_axis_name="s",
                                  num_cores=NC, num_subcores=NS)
    nch2 = nchunks // 2

    def body(table, src, dst, zh, z1, *rest):
        if with_deg:
            out, degout, sidx, didx, rows, ones, acc, dacc, tab = rest[:9]
        else:
            out, sidx, didx, rows, ones, acc, dacc, tab = rest[:8]
            degout = None
        sems = rest[-2 * NPOOL:]
        c = lax.axis_index("c")
        s = lax.axis_index("s")

        for i in range(CH // 16):
            ones[pl.ds(i * 16, 16)] = jnp.ones((16,), jnp.float32)

        @pl.when(s == 0)
        def _():
            pltpu.sync_copy(zh, acc)
            pltpu.sync_copy(z1, dacc)
            pltpu.sync_copy(table.at[c], tab)

        pltpu.sync_copy(src.at[pl.ds(s * nchunks, nchunks)], sidx)
        pltpu.sync_copy(dst.at[pl.ds(s * nchunks, nchunks)], didx)
        plsc.subcore_barrier()

        def sidx_of(j):
            return sidx.at[j]

        def didx_of(j):
            return didx.at[j]

        for b in range(NB):
            pltpu.async_copy(tab.at[sidx_of(b)], rows.at[b], sems[b])

        def pair(p, carry):
            for u in range(NPOOL):
                j = p * NPOOL + u
                b = u
                bn = (u + NB) % NPOOL
                pltpu.make_async_copy(tab.at[sidx_of(j)], rows.at[b],
                                      sems[b]).wait()
                pltpu.async_copy(rows.at[b], acc.at[didx_of(j)],
                                 sems[NPOOL + b], add=True)
                if with_deg:
                    @pl.when((j < nch2) == (c == 0))
                    def _():
                        pltpu.sync_copy(ones, dacc.at[didx_of(j)], add=True)

                @pl.when(jnp.logical_and(j + NB < nchunks, j >= NB))
                def _():
                    pltpu.make_async_copy(rows.at[bn], acc.at[didx_of(0)],
                                          sems[NPOOL + bn]).wait()

                @pl.when(j + NB < nchunks)
                def _():
                    pltpu.async_copy(tab.at[sidx_of(j + NB)], rows.at[bn],
                                     sems[bn])
            return carry

        lax.fori_loop(0, nchunks // NPOOL, pair, 0)

        for b in range(NPOOL):
            pltpu.make_async_copy(rows.at[b], acc.at[didx_of(0)],
                                  sems[NPOOL + b]).wait()
        plsc.subcore_barrier()

        r0 = s * ROWS_PER_TILE
        pltpu.sync_copy(acc.at[pl.ds(r0, ROWS_PER_TILE)],
                        out.at[c, pl.ds(r0, ROWS_PER_TILE)])
        if with_deg:
            pltpu.sync_copy(dacc.at[pl.ds(r0, ROWS_PER_TILE)],
                            degout.at[c, pl.ds(r0, ROWS_PER_TILE)])

    return pl.kernel(body, out_type=out_type, mesh=mesh, scratch_types=scratch,
                     compiler_params=pltpu.CompilerParams(use_tc_tiling_on_sc=False))


def _tc1_body(x_ref, wl_ref, wr_ref, y_ref, r_ref):
    x = x_ref[...]
    y = jnp.dot(x, wl_ref[...], preferred_element_type=jnp.float32)
    y_ref[0] = y[:, :HH]
    y_ref[1] = y[:, HH:]
    r_ref[...] = jnp.dot(x, wr_ref[...], preferred_element_type=jnp.float32)


def _tc2_body(p_ref, dpt_ref, r1_ref, b1_ref, wl_ref, wr_ref, y_ref, r_ref):
    agg = jnp.concatenate([p_ref[0, :N], p_ref[1, :N]], axis=-1)
    deg = dpt_ref[:N, 0] + dpt_ref[:N, 1]
    inv = 1.0 / jnp.maximum(deg, 1.0)
    h = jnp.maximum(agg * inv[:, None] + r1_ref[...] + b1_ref[...][None, :], 0.0)
    y = jnp.dot(h, wl_ref[...], preferred_element_type=jnp.float32)
    y_ref[0] = y[:, :HH]
    y_ref[1] = y[:, HH:]
    r_ref[...] = jnp.dot(h, wr_ref[...], preferred_element_type=jnp.float32)


def _tc3_body(p_ref, dpt_ref, r2_ref, b2_ref, wc_ref, bc_ref, log_ref, emb_ref):
    agg = jnp.concatenate([p_ref[0, :N], p_ref[1, :N]], axis=-1)
    deg = dpt_ref[:N, 0] + dpt_ref[:N, 1]
    inv = 1.0 / jnp.maximum(deg, 1.0)
    emb = agg * inv[:, None] + r2_ref[...] + b2_ref[...][None, :]
    emb_ref[...] = emb
    log_ref[...] = (jnp.dot(emb, wc_ref[...], preferred_element_type=jnp.float32)
                    + bc_ref[...][None, :])


def _whole(shape):
    return pl.BlockSpec(shape, lambda i: tuple(0 for _ in shape))


_tc1 = pl.pallas_call(
    _tc1_body,
    grid=(1,),
    in_specs=[_whole((N, IN)), _whole((IN, H)), _whole((IN, H))],
    out_specs=[_whole((NC, N, HH)), _whole((N, H))],
    out_shape=[jax.ShapeDtypeStruct((NC, N, HH), jnp.float32),
               jax.ShapeDtypeStruct((N, H), jnp.float32)],
)

_tc2 = pl.pallas_call(
    _tc2_body,
    grid=(1,),
    in_specs=[
        _whole((NC, NPAD, HH)),
        _whole((NPAD, NC)),
        _whole((N, H)),
        _whole((H,)),
        _whole((H, H)),
        _whole((H, H)),
    ],
    out_specs=[_whole((NC, N, HH)), _whole((N, H))],
    out_shape=[jax.ShapeDtypeStruct((NC, N, HH), jnp.float32),
               jax.ShapeDtypeStruct((N, H), jnp.float32)],
)

_tc3 = pl.pallas_call(
    _tc3_body,
    grid=(1,),
    in_specs=[
        _whole((NC, NPAD, HH)),
        _whole((NPAD, NC)),
        _whole((N, H)),
        _whole((H,)),
        _whole((H, C)),
        _whole((C,)),
    ],
    out_specs=[_whole((N, C)), _whole((N, H))],
    out_shape=[jax.ShapeDtypeStruct((N, C), jnp.float32),
               jax.ShapeDtypeStruct((N, H), jnp.float32)],
)


def _make_edge_prep(E: int, e_pad: int):
    nrows = e_pad // CH

    def body(ei_ref, srcm_ref, dstm_ref):
        sm = ei_ref[0].reshape(nrows, CH)
        dm = ei_ref[1].reshape(nrows, CH)
        flat = (lax.broadcasted_iota(jnp.int32, (nrows, CH), 0) * CH +
                lax.broadcasted_iota(jnp.int32, (nrows, CH), 1))
        mask = flat < E
        srcm_ref[...] = jnp.where(mask, sm, 0)
        dstm_ref[...] = jnp.where(mask, dm, NPAD - 1)

    return pl.pallas_call(
        body,
        grid=(1,),
        in_specs=[pl.BlockSpec((2, e_pad), lambda i: (0, 0))],
        out_specs=[_whole((nrows, CH)), _whole((nrows, CH))],
        out_shape=[jax.ShapeDtypeStruct((nrows, CH), jnp.int32)] * 2,
    )


def kernel(x, edge_index, W1l, b1, W1r, W2l, b2, W2r, Wc, bc):
    E = edge_index.shape[1]
    nchunks = -(-(-(-E // (NS * CH))) // NPOOL) * NPOOL
    e_pad = NS * nchunks * CH

    srcm, dstm = _make_edge_prep(E, e_pad)(edge_index)

    zh = jnp.zeros((NPAD, HH), jnp.float32)
    z1 = jnp.zeros((NPAD,), jnp.float32)

    sc_agg1 = _make_sc_agg(nchunks, with_deg=True)
    sc_agg2 = _make_sc_agg(nchunks, with_deg=False)

    y1, r1 = _tc1(x, W1l, W1r)
    p1, dp = sc_agg1(y1, srcm, dstm, zh, z1)
    dpt = dp.T
    y2, r2 = _tc2(p1, dpt, r1, b1, W2l, W2r)
    p2 = sc_agg2(y2, srcm, dstm, zh, z1)
    p2 = p2[0] if isinstance(p2, (list, tuple)) else p2
    logits, emb = _tc3(p2, dpt, r2, b2, Wc, bc)
    return logits, emb


# speedup vs baseline: 2.5672x; 1.0946x over previous
import functools

import jax
import jax.numpy as jnp
from jax import lax
from jax.experimental import pallas as pl
from jax.experimental.pallas import tpu as pltpu
from jax.experimental.pallas import tpu_sc as plsc

N = 10000
IN = 128
H = 64
C = 32

NPAD = 10240
CH = 128
NC = 2
NS = 16
NW = NC * NS
ROWS_PER_TILE = NPAD // NS
NB = 5
NPOOL = 2 * NB


HH = H // NC


def _make_sc_agg(nchunks: int, with_deg: bool):
    out_type = [jax.ShapeDtypeStruct((NC, NPAD, 128), jnp.float32)]
    if with_deg:
        out_type.append(jax.ShapeDtypeStruct((NC, NPAD), jnp.float32))

    scratch = [
        pltpu.VMEM((nchunks, CH), jnp.int32),
        pltpu.VMEM((nchunks, CH), jnp.int32),
        pltpu.VMEM((NPOOL, CH, HH), jnp.float32),
        pltpu.VMEM((CH,), jnp.float32),
        pltpu.VMEM_SHARED((NPAD, HH), jnp.float32),
        pltpu.VMEM_SHARED((NPAD,), jnp.float32),
        pltpu.VMEM_SHARED((N, HH), jnp.float32),
    ] + [pltpu.SemaphoreType.DMA] * (2 * NPOOL)

    mesh = plsc.VectorSubcoreMesh(core_axis_name="c", subcore_axis_name="s",
                                  num_cores=NC, num_subcores=NS)
    nch2 = nchunks // 2

    def body(table, src, dst, zh, z1, *rest):
        if with_deg:
            out, degout, sidx, didx, rows, ones, acc, dacc, tab = rest[:9]
        else:
            out, sidx, didx, rows, ones, acc, dacc, tab = rest[:8]
            degout = None
        sems = rest[-2 * NPOOL:]
        c = lax.axis_index("c")
        s = lax.axis_index("s")

        for i in range(CH // 16):
            ones[pl.ds(i * 16, 16)] = jnp.ones((16,), jnp.float32)

        @pl.when(s == 0)
        def _():
            pltpu.sync_copy(zh, acc)
            pltpu.sync_copy(z1, dacc)
            pltpu.sync_copy(table.at[c, :, pl.ds(0, HH)], tab)

        pltpu.sync_copy(src.at[pl.ds(s * nchunks, nchunks)], sidx)
        pltpu.sync_copy(dst.at[pl.ds(s * nchunks, nchunks)], didx)
        plsc.subcore_barrier()

        def sidx_of(j):
            return sidx.at[j]

        def didx_of(j):
            return didx.at[j]

        for b in range(NB):
            pltpu.async_copy(tab.at[sidx_of(b)], rows.at[b], sems[b])

        def pair(p, carry):
            for u in range(NPOOL):
                j = p * NPOOL + u
                b = u
                bn = (u + NB) % NPOOL
                pltpu.make_async_copy(tab.at[sidx_of(j)], rows.at[b],
                                      sems[b]).wait()
                pltpu.async_copy(rows.at[b], acc.at[didx_of(j)],
                                 sems[NPOOL + b], add=True)
                if with_deg:
                    @pl.when((j < nch2) == (c == 0))
                    def _():
                        pltpu.sync_copy(ones, dacc.at[didx_of(j)], add=True)

                @pl.when(jnp.logical_and(j + NB < nchunks, j >= NB))
                def _():
                    pltpu.make_async_copy(rows.at[bn], acc.at[didx_of(0)],
                                          sems[NPOOL + bn]).wait()

                @pl.when(j + NB < nchunks)
                def _():
                    pltpu.async_copy(tab.at[sidx_of(j + NB)], rows.at[bn],
                                     sems[bn])
            return carry

        lax.fori_loop(0, nchunks // NPOOL, pair, 0)

        for b in range(NPOOL):
            pltpu.make_async_copy(rows.at[b], acc.at[didx_of(0)],
                                  sems[NPOOL + b]).wait()
        plsc.subcore_barrier()

        r0 = s * ROWS_PER_TILE
        pltpu.sync_copy(acc.at[pl.ds(r0, ROWS_PER_TILE)],
                        out.at[c, pl.ds(r0, ROWS_PER_TILE), pl.ds(0, HH)])
        if with_deg:
            pltpu.sync_copy(dacc.at[pl.ds(r0, ROWS_PER_TILE)],
                            degout.at[c, pl.ds(r0, ROWS_PER_TILE)])

    return pl.kernel(body, out_type=out_type, mesh=mesh, scratch_types=scratch,
                     compiler_params=pltpu.CompilerParams(use_tc_tiling_on_sc=False))


def _pad128(y):
    return jnp.concatenate([y, jnp.zeros((y.shape[0], 128 - HH), y.dtype)],
                           axis=-1)


def _tc1_body(x_ref, wl_ref, wr_ref, y_ref, r_ref):
    x = x_ref[...]
    y = jnp.dot(x, wl_ref[...], preferred_element_type=jnp.float32)
    y_ref[0] = _pad128(y[:, :HH])
    y_ref[1] = _pad128(y[:, HH:])
    r_ref[...] = jnp.dot(x, wr_ref[...], preferred_element_type=jnp.float32)


def _tc2_body(p_ref, dpt_ref, r1_ref, b1_ref, wl_ref, wr_ref, y_ref, r_ref):
    agg = jnp.concatenate([p_ref[0, :N, :HH], p_ref[1, :N, :HH]], axis=-1)
    deg = dpt_ref[:N, 0] + dpt_ref[:N, 1]
    inv = 1.0 / jnp.maximum(deg, 1.0)
    h = jnp.maximum(agg * inv[:, None] + r1_ref[...] + b1_ref[...][None, :], 0.0)
    y = jnp.dot(h, wl_ref[...], preferred_element_type=jnp.float32)
    y_ref[0] = _pad128(y[:, :HH])
    y_ref[1] = _pad128(y[:, HH:])
    r_ref[...] = jnp.dot(h, wr_ref[...], preferred_element_type=jnp.float32)


def _tc3_body(p_ref, dpt_ref, r2_ref, b2_ref, wc_ref, bc_ref, log_ref, emb_ref):
    agg = jnp.concatenate([p_ref[0, :N, :HH], p_ref[1, :N, :HH]], axis=-1)
    deg = dpt_ref[:N, 0] + dpt_ref[:N, 1]
    inv = 1.0 / jnp.maximum(deg, 1.0)
    emb = agg * inv[:, None] + r2_ref[...] + b2_ref[...][None, :]
    emb_ref[...] = emb
    log_ref[...] = (jnp.dot(emb, wc_ref[...], preferred_element_type=jnp.float32)
                    + bc_ref[...][None, :])


def _whole(shape):
    return pl.BlockSpec(shape, lambda i: tuple(0 for _ in shape))


_tc1 = pl.pallas_call(
    _tc1_body,
    grid=(1,),
    in_specs=[_whole((N, IN)), _whole((IN, H)), _whole((IN, H))],
    out_specs=[_whole((NC, N, 128)), _whole((N, H))],
    out_shape=[jax.ShapeDtypeStruct((NC, N, 128), jnp.float32),
               jax.ShapeDtypeStruct((N, H), jnp.float32)],
)

_tc2 = pl.pallas_call(
    _tc2_body,
    grid=(1,),
    in_specs=[
        _whole((NC, NPAD, 128)),
        _whole((NPAD, NC)),
        _whole((N, H)),
        _whole((H,)),
        _whole((H, H)),
        _whole((H, H)),
    ],
    out_specs=[_whole((NC, N, 128)), _whole((N, H))],
    out_shape=[jax.ShapeDtypeStruct((NC, N, 128), jnp.float32),
               jax.ShapeDtypeStruct((N, H), jnp.float32)],
)

_tc3 = pl.pallas_call(
    _tc3_body,
    grid=(1,),
    in_specs=[
        _whole((NC, NPAD, 128)),
        _whole((NPAD, NC)),
        _whole((N, H)),
        _whole((H,)),
        _whole((H, C)),
        _whole((C,)),
    ],
    out_specs=[_whole((N, C)), _whole((N, H))],
    out_shape=[jax.ShapeDtypeStruct((N, C), jnp.float32),
               jax.ShapeDtypeStruct((N, H), jnp.float32)],
)


def _make_edge_prep(E: int, e_pad: int):
    nrows = e_pad // CH

    def body(ei_ref, srcm_ref, dstm_ref):
        sm = ei_ref[0].reshape(nrows, CH)
        dm = ei_ref[1].reshape(nrows, CH)
        flat = (lax.broadcasted_iota(jnp.int32, (nrows, CH), 0) * CH +
                lax.broadcasted_iota(jnp.int32, (nrows, CH), 1))
        mask = flat < E
        srcm_ref[...] = jnp.where(mask, sm, 0)
        dstm_ref[...] = jnp.where(mask, dm, NPAD - 1)

    return pl.pallas_call(
        body,
        grid=(1,),
        in_specs=[pl.BlockSpec((2, e_pad), lambda i: (0, 0))],
        out_specs=[_whole((nrows, CH)), _whole((nrows, CH))],
        out_shape=[jax.ShapeDtypeStruct((nrows, CH), jnp.int32)] * 2,
    )


def kernel(x, edge_index, W1l, b1, W1r, W2l, b2, W2r, Wc, bc):
    E = edge_index.shape[1]
    nchunks = -(-(-(-E // (NS * CH))) // NPOOL) * NPOOL
    e_pad = NS * nchunks * CH

    srcm, dstm = _make_edge_prep(E, e_pad)(edge_index)

    zh = jnp.zeros((NPAD, HH), jnp.float32)
    z1 = jnp.zeros((NPAD,), jnp.float32)

    sc_agg1 = _make_sc_agg(nchunks, with_deg=True)
    sc_agg2 = _make_sc_agg(nchunks, with_deg=False)

    y1, r1 = _tc1(x, W1l, W1r)
    p1, dp = sc_agg1(y1, srcm, dstm, zh, z1)
    dpt = dp.T
    y2, r2 = _tc2(p1, dpt, r1, b1, W2l, W2r)
    p2 = sc_agg2(y2, srcm, dstm, zh, z1)
    p2 = p2[0] if isinstance(p2, (list, tuple)) else p2
    logits, emb = _tc3(p2, dpt, r2, b2, Wc, bc)
    return logits, emb


# speedup vs baseline: 2.5958x; 1.0112x over previous
import functools

import jax
import jax.numpy as jnp
from jax import lax
from jax.experimental import pallas as pl
from jax.experimental.pallas import tpu as pltpu
from jax.experimental.pallas import tpu_sc as plsc

N = 10000
IN = 128
H = 64
C = 32

NPAD = 10240
CH = 128
NC = 2
NS = 16
NW = NC * NS
ROWS_PER_TILE = NPAD // NS
NB = 5
NPOOL = 2 * NB


HH = H // NC


def _make_sc_agg(nchunks: int, with_deg: bool):
    out_type = [jax.ShapeDtypeStruct((NC, NPAD, 128), jnp.float32)]
    if with_deg:
        out_type.append(jax.ShapeDtypeStruct((NC, NPAD), jnp.float32))

    scratch = [
        pltpu.VMEM((nchunks, CH), jnp.int32),
        pltpu.VMEM((nchunks, CH), jnp.int32),
        pltpu.VMEM((NPOOL, CH, HH), jnp.float32),
        pltpu.VMEM((CH,), jnp.float32),
        pltpu.VMEM_SHARED((NPAD, HH), jnp.float32),
        pltpu.VMEM_SHARED((NPAD,), jnp.float32),
        pltpu.VMEM_SHARED((N, HH), jnp.float32),
    ] + [pltpu.SemaphoreType.DMA] * (2 * NPOOL)

    mesh = plsc.VectorSubcoreMesh(core_axis_name="c", subcore_axis_name="s",
                                  num_cores=NC, num_subcores=NS)
    nch2 = nchunks // 2

    def body(table, src, dst, zh, z1, *rest):
        if with_deg:
            out, degout, sidx, didx, rows, ones, acc, dacc, tab = rest[:9]
        else:
            out, sidx, didx, rows, ones, acc, dacc, tab = rest[:8]
            degout = None
        sems = rest[-2 * NPOOL:]
        c = lax.axis_index("c")
        s = lax.axis_index("s")

        for i in range(CH // 16):
            ones[pl.ds(i * 16, 16)] = jnp.ones((16,), jnp.float32)

        r0 = s * ROWS_PER_TILE
        pltpu.sync_copy(zh.at[pl.ds(r0, ROWS_PER_TILE)],
                        acc.at[pl.ds(r0, ROWS_PER_TILE)])
        pltpu.sync_copy(z1.at[pl.ds(r0, ROWS_PER_TILE)],
                        dacc.at[pl.ds(r0, ROWS_PER_TILE)])
        tr = N // NS
        pltpu.sync_copy(table.at[c, pl.ds(s * tr, tr), pl.ds(0, HH)],
                        tab.at[pl.ds(s * tr, tr)])

        pltpu.sync_copy(src.at[pl.ds(s * nchunks, nchunks)], sidx)
        pltpu.sync_copy(dst.at[pl.ds(s * nchunks, nchunks)], didx)
        plsc.subcore_barrier()

        def sidx_of(j):
            return sidx.at[j]

        def didx_of(j):
            return didx.at[j]

        for b in range(NB):
            pltpu.async_copy(tab.at[sidx_of(b)], rows.at[b], sems[b])

        def pair(p, carry):
            for u in range(NPOOL):
                j = p * NPOOL + u
                b = u
                bn = (u + NB) % NPOOL
                pltpu.make_async_copy(tab.at[sidx_of(j)], rows.at[b],
                                      sems[b]).wait()
                pltpu.async_copy(rows.at[b], acc.at[didx_of(j)],
                                 sems[NPOOL + b], add=True)
                if with_deg:
                    @pl.when((j < nch2) == (c == 0))
                    def _():
                        pltpu.sync_copy(ones, dacc.at[didx_of(j)], add=True)

                @pl.when(jnp.logical_and(j + NB < nchunks, j >= NB))
                def _():
                    pltpu.make_async_copy(rows.at[bn], acc.at[didx_of(0)],
                                          sems[NPOOL + bn]).wait()

                @pl.when(j + NB < nchunks)
                def _():
                    pltpu.async_copy(tab.at[sidx_of(j + NB)], rows.at[bn],
                                     sems[bn])
            return carry

        lax.fori_loop(0, nchunks // NPOOL, pair, 0)

        for b in range(NPOOL):
            pltpu.make_async_copy(rows.at[b], acc.at[didx_of(0)],
                                  sems[NPOOL + b]).wait()
        plsc.subcore_barrier()

        pltpu.sync_copy(acc.at[pl.ds(r0, ROWS_PER_TILE)],
                        out.at[c, pl.ds(r0, ROWS_PER_TILE), pl.ds(0, HH)])
        if with_deg:
            pltpu.sync_copy(dacc.at[pl.ds(r0, ROWS_PER_TILE)],
                            degout.at[c, pl.ds(r0, ROWS_PER_TILE)])

    return pl.kernel(body, out_type=out_type, mesh=mesh, scratch_types=scratch,
                     compiler_params=pltpu.CompilerParams(use_tc_tiling_on_sc=False))


def _pad128(y):
    return jnp.concatenate([y, jnp.zeros((y.shape[0], 128 - HH), y.dtype)],
                           axis=-1)


def _tc1_body(x_ref, wl_ref, wr_ref, y_ref, r_ref):
    x = x_ref[...]
    y = jnp.dot(x, wl_ref[...], preferred_element_type=jnp.float32)
    y_ref[0] = _pad128(y[:, :HH])
    y_ref[1] = _pad128(y[:, HH:])
    r_ref[...] = jnp.dot(x, wr_ref[...], preferred_element_type=jnp.float32)


def _tc2_body(p_ref, dpt_ref, r1_ref, b1_ref, wl_ref, wr_ref, y_ref, r_ref):
    agg = jnp.concatenate([p_ref[0, :N, :HH], p_ref[1, :N, :HH]], axis=-1)
    deg = dpt_ref[:N, 0] + dpt_ref[:N, 1]
    inv = 1.0 / jnp.maximum(deg, 1.0)
    h = jnp.maximum(agg * inv[:, None] + r1_ref[...] + b1_ref[...][None, :], 0.0)
    y = jnp.dot(h, wl_ref[...], preferred_element_type=jnp.float32)
    y_ref[0] = _pad128(y[:, :HH])
    y_ref[1] = _pad128(y[:, HH:])
    r_ref[...] = jnp.dot(h, wr_ref[...], preferred_element_type=jnp.float32)


def _tc3_body(p_ref, dpt_ref, r2_ref, b2_ref, wc_ref, bc_ref, log_ref, emb_ref):
    agg = jnp.concatenate([p_ref[0, :N, :HH], p_ref[1, :N, :HH]], axis=-1)
    deg = dpt_ref[:N, 0] + dpt_ref[:N, 1]
    inv = 1.0 / jnp.maximum(deg, 1.0)
    emb = agg * inv[:, None] + r2_ref[...] + b2_ref[...][None, :]
    emb_ref[...] = emb
    log_ref[...] = (jnp.dot(emb, wc_ref[...], preferred_element_type=jnp.float32)
                    + bc_ref[...][None, :])


def _whole(shape):
    return pl.BlockSpec(shape, lambda i: tuple(0 for _ in shape))


_tc1 = pl.pallas_call(
    _tc1_body,
    grid=(1,),
    in_specs=[_whole((N, IN)), _whole((IN, H)), _whole((IN, H))],
    out_specs=[_whole((NC, N, 128)), _whole((N, H))],
    out_shape=[jax.ShapeDtypeStruct((NC, N, 128), jnp.float32),
               jax.ShapeDtypeStruct((N, H), jnp.float32)],
)

_tc2 = pl.pallas_call(
    _tc2_body,
    grid=(1,),
    in_specs=[
        _whole((NC, NPAD, 128)),
        _whole((NPAD, NC)),
        _whole((N, H)),
        _whole((H,)),
        _whole((H, H)),
        _whole((H, H)),
    ],
    out_specs=[_whole((NC, N, 128)), _whole((N, H))],
    out_shape=[jax.ShapeDtypeStruct((NC, N, 128), jnp.float32),
               jax.ShapeDtypeStruct((N, H), jnp.float32)],
)

_tc3 = pl.pallas_call(
    _tc3_body,
    grid=(1,),
    in_specs=[
        _whole((NC, NPAD, 128)),
        _whole((NPAD, NC)),
        _whole((N, H)),
        _whole((H,)),
        _whole((H, C)),
        _whole((C,)),
    ],
    out_specs=[_whole((N, C)), _whole((N, H))],
    out_shape=[jax.ShapeDtypeStruct((N, C), jnp.float32),
               jax.ShapeDtypeStruct((N, H), jnp.float32)],
)


def _make_edge_prep(E: int, e_pad: int):
    nrows = e_pad // CH

    def body(ei_ref, srcm_ref, dstm_ref):
        sm = ei_ref[0].reshape(nrows, CH)
        dm = ei_ref[1].reshape(nrows, CH)
        flat = (lax.broadcasted_iota(jnp.int32, (nrows, CH), 0) * CH +
                lax.broadcasted_iota(jnp.int32, (nrows, CH), 1))
        mask = flat < E
        srcm_ref[...] = jnp.where(mask, sm, 0)
        dstm_ref[...] = jnp.where(mask, dm, NPAD - 1)

    return pl.pallas_call(
        body,
        grid=(1,),
        in_specs=[pl.BlockSpec((2, e_pad), lambda i: (0, 0))],
        out_specs=[_whole((nrows, CH)), _whole((nrows, CH))],
        out_shape=[jax.ShapeDtypeStruct((nrows, CH), jnp.int32)] * 2,
    )


def kernel(x, edge_index, W1l, b1, W1r, W2l, b2, W2r, Wc, bc):
    E = edge_index.shape[1]
    nchunks = -(-(-(-E // (NS * CH))) // NPOOL) * NPOOL
    e_pad = NS * nchunks * CH

    srcm, dstm = _make_edge_prep(E, e_pad)(edge_index)

    zh = jnp.zeros((NPAD, HH), jnp.float32)
    z1 = jnp.zeros((NPAD,), jnp.float32)

    sc_agg1 = _make_sc_agg(nchunks, with_deg=True)
    sc_agg2 = _make_sc_agg(nchunks, with_deg=False)

    y1, r1 = _tc1(x, W1l, W1r)
    p1, dp = sc_agg1(y1, srcm, dstm, zh, z1)
    dpt = dp.T
    y2, r2 = _tc2(p1, dpt, r1, b1, W2l, W2r)
    p2 = sc_agg2(y2, srcm, dstm, zh, z1)
    p2 = p2[0] if isinstance(p2, (list, tuple)) else p2
    logits, emb = _tc3(p2, dpt, r2, b2, Wc, bc)
    return logits, emb


# speedup vs baseline: 2.6276x; 1.0123x over previous
import functools

import jax
import jax.numpy as jnp
from jax import lax
from jax.experimental import pallas as pl
from jax.experimental.pallas import tpu as pltpu
from jax.experimental.pallas import tpu_sc as plsc

N = 10000
IN = 128
H = 64
C = 32

NPAD = 10240
CH = 128
NC = 2
NS = 16
NW = NC * NS
ROWS_PER_TILE = NPAD // NS
NB = 5
NPOOL = 2 * NB


HH = H // NC


def _make_sc_agg(nchunks: int, with_deg: bool):
    out_type = [jax.ShapeDtypeStruct((NC, NPAD, 128), jnp.float32)]
    if with_deg:
        out_type.append(jax.ShapeDtypeStruct((NC, NPAD), jnp.float32))

    scratch = [
        pltpu.VMEM((nchunks, CH), jnp.int32),
        pltpu.VMEM((nchunks, CH), jnp.int32),
        pltpu.VMEM((NPOOL, CH, HH), jnp.float32),
        pltpu.VMEM((CH,), jnp.float32),
        pltpu.VMEM_SHARED((NPAD, HH), jnp.float32),
        pltpu.VMEM_SHARED((NPAD,), jnp.float32),
        pltpu.VMEM_SHARED((N, HH), jnp.float32),
    ] + [pltpu.SemaphoreType.DMA] * (2 * NPOOL)

    mesh = plsc.VectorSubcoreMesh(core_axis_name="c", subcore_axis_name="s",
                                  num_cores=NC, num_subcores=NS)
    nch2 = nchunks // 2

    def body(table, src, dst, zh, z1, *rest):
        if with_deg:
            out, degout, sidx, didx, rows, ones, acc, dacc, tab = rest[:9]
        else:
            out, sidx, didx, rows, ones, acc, dacc, tab = rest[:8]
            degout = None
        sems = rest[-2 * NPOOL:]
        c = lax.axis_index("c")
        s = lax.axis_index("s")

        for i in range(CH // 16):
            ones[pl.ds(i * 16, 16)] = jnp.ones((16,), jnp.float32)

        r0 = s * ROWS_PER_TILE
        pltpu.sync_copy(zh.at[pl.ds(r0, ROWS_PER_TILE)],
                        acc.at[pl.ds(r0, ROWS_PER_TILE)])
        pltpu.sync_copy(z1.at[pl.ds(r0, ROWS_PER_TILE)],
                        dacc.at[pl.ds(r0, ROWS_PER_TILE)])
        tr = N // NS

        @pl.when(c == 0)
        def _():
            pltpu.sync_copy(table.at[pl.ds(s * tr, tr), pl.ds(0, HH)],
                            tab.at[pl.ds(s * tr, tr)])

        @pl.when(c == 1)
        def _():
            pltpu.sync_copy(table.at[pl.ds(s * tr, tr), pl.ds(HH, HH)],
                            tab.at[pl.ds(s * tr, tr)])

        pltpu.sync_copy(src.at[pl.ds(s * nchunks, nchunks)], sidx)
        pltpu.sync_copy(dst.at[pl.ds(s * nchunks, nchunks)], didx)
        plsc.subcore_barrier()

        def sidx_of(j):
            return sidx.at[j]

        def didx_of(j):
            return didx.at[j]

        for b in range(NB):
            pltpu.async_copy(tab.at[sidx_of(b)], rows.at[b], sems[b])

        def pair(p, carry):
            for u in range(NPOOL):
                j = p * NPOOL + u
                b = u
                bn = (u + NB) % NPOOL
                pltpu.make_async_copy(tab.at[sidx_of(j)], rows.at[b],
                                      sems[b]).wait()
                pltpu.async_copy(rows.at[b], acc.at[didx_of(j)],
                                 sems[NPOOL + b], add=True)
                if with_deg:
                    @pl.when((j < nch2) == (c == 0))
                    def _():
                        pltpu.sync_copy(ones, dacc.at[didx_of(j)], add=True)

                @pl.when(jnp.logical_and(j + NB < nchunks, j >= NB))
                def _():
                    pltpu.make_async_copy(rows.at[bn], acc.at[didx_of(0)],
                                          sems[NPOOL + bn]).wait()

                @pl.when(j + NB < nchunks)
                def _():
                    pltpu.async_copy(tab.at[sidx_of(j + NB)], rows.at[bn],
                                     sems[bn])
            return carry

        lax.fori_loop(0, nchunks // NPOOL, pair, 0)

        for b in range(NPOOL):
            pltpu.make_async_copy(rows.at[b], acc.at[didx_of(0)],
                                  sems[NPOOL + b]).wait()
        plsc.subcore_barrier()

        pltpu.sync_copy(acc.at[pl.ds(r0, ROWS_PER_TILE)],
                        out.at[c, pl.ds(r0, ROWS_PER_TILE), pl.ds(0, HH)])
        if with_deg:
            pltpu.sync_copy(dacc.at[pl.ds(r0, ROWS_PER_TILE)],
                            degout.at[c, pl.ds(r0, ROWS_PER_TILE)])

    return pl.kernel(body, out_type=out_type, mesh=mesh, scratch_types=scratch,
                     compiler_params=pltpu.CompilerParams(use_tc_tiling_on_sc=False))


def _tc1_body(x_ref, wl_ref, wr_ref, a_ref):
    x = x_ref[...]
    y = jnp.dot(x, wl_ref[...], preferred_element_type=jnp.float32)
    r = jnp.dot(x, wr_ref[...], preferred_element_type=jnp.float32)
    a_ref[...] = jnp.concatenate([y, r], axis=-1)


def _tc2_body(p_ref, dpt_ref, a1_ref, b1_ref, wl_ref, wr_ref, a_ref):
    agg = jnp.concatenate([p_ref[0, :N, :HH], p_ref[1, :N, :HH]], axis=-1)
    deg = dpt_ref[:N, 0] + dpt_ref[:N, 1]
    inv = 1.0 / jnp.maximum(deg, 1.0)
    r1 = a1_ref[:, H:]
    h = jnp.maximum(agg * inv[:, None] + r1 + b1_ref[...][None, :], 0.0)
    y = jnp.dot(h, wl_ref[...], preferred_element_type=jnp.float32)
    r = jnp.dot(h, wr_ref[...], preferred_element_type=jnp.float32)
    a_ref[...] = jnp.concatenate([y, r], axis=-1)


def _tc3_body(p_ref, dpt_ref, a2_ref, b2_ref, wc_ref, bc_ref, log_ref, emb_ref):
    agg = jnp.concatenate([p_ref[0, :N, :HH], p_ref[1, :N, :HH]], axis=-1)
    deg = dpt_ref[:N, 0] + dpt_ref[:N, 1]
    inv = 1.0 / jnp.maximum(deg, 1.0)
    emb = agg * inv[:, None] + a2_ref[:, H:] + b2_ref[...][None, :]
    emb_ref[...] = emb
    log_ref[...] = (jnp.dot(emb, wc_ref[...], preferred_element_type=jnp.float32)
                    + bc_ref[...][None, :])


def _whole(shape):
    return pl.BlockSpec(shape, lambda i: tuple(0 for _ in shape))


_tc1 = pl.pallas_call(
    _tc1_body,
    grid=(1,),
    in_specs=[_whole((N, IN)), _whole((IN, H)), _whole((IN, H))],
    out_specs=[_whole((N, 2 * H))],
    out_shape=[jax.ShapeDtypeStruct((N, 2 * H), jnp.float32)],
)

_tc2 = pl.pallas_call(
    _tc2_body,
    grid=(1,),
    in_specs=[
        _whole((NC, NPAD, 128)),
        _whole((NPAD, NC)),
        _whole((N, 2 * H)),
        _whole((H,)),
        _whole((H, H)),
        _whole((H, H)),
    ],
    out_specs=[_whole((N, 2 * H))],
    out_shape=[jax.ShapeDtypeStruct((N, 2 * H), jnp.float32)],
)

_tc3 = pl.pallas_call(
    _tc3_body,
    grid=(1,),
    in_specs=[
        _whole((NC, NPAD, 128)),
        _whole((NPAD, NC)),
        _whole((N, 2 * H)),
        _whole((H,)),
        _whole((H, C)),
        _whole((C,)),
    ],
    out_specs=[_whole((N, C)), _whole((N, H))],
    out_shape=[jax.ShapeDtypeStruct((N, C), jnp.float32),
               jax.ShapeDtypeStruct((N, H), jnp.float32)],
)


def _make_edge_prep(E: int, e_pad: int):
    nrows = e_pad // CH

    def body(ei_ref, srcm_ref, dstm_ref):
        sm = ei_ref[0].reshape(nrows, CH)
        dm = ei_ref[1].reshape(nrows, CH)
        flat = (lax.broadcasted_iota(jnp.int32, (nrows, CH), 0) * CH +
                lax.broadcasted_iota(jnp.int32, (nrows, CH), 1))
        mask = flat < E
        srcm_ref[...] = jnp.where(mask, sm, 0)
        dstm_ref[...] = jnp.where(mask, dm, NPAD - 1)

    return pl.pallas_call(
        body,
        grid=(1,),
        in_specs=[pl.BlockSpec((2, e_pad), lambda i: (0, 0))],
        out_specs=[_whole((nrows, CH)), _whole((nrows, CH))],
        out_shape=[jax.ShapeDtypeStruct((nrows, CH), jnp.int32)] * 2,
    )


def kernel(x, edge_index, W1l, b1, W1r, W2l, b2, W2r, Wc, bc):
    E = edge_index.shape[1]
    nchunks = -(-(-(-E // (NS * CH))) // NPOOL) * NPOOL
    e_pad = NS * nchunks * CH

    srcm, dstm = _make_edge_prep(E, e_pad)(edge_index)

    zh = jnp.zeros((NPAD, HH), jnp.float32)
    z1 = jnp.zeros((NPAD,), jnp.float32)

    sc_agg1 = _make_sc_agg(nchunks, with_deg=True)
    sc_agg2 = _make_sc_agg(nchunks, with_deg=False)

    (a1,) = _tc1(x, W1l, W1r)
    p1, dp = sc_agg1(a1, srcm, dstm, zh, z1)
    dpt = dp.T
    (a2,) = _tc2(p1, dpt, a1, b1, W2l, W2r)
    p2 = sc_agg2(a2, srcm, dstm, zh, z1)
    p2 = p2[0] if isinstance(p2, (list, tuple)) else p2
    logits, emb = _tc3(p2, dpt, a2, b2, Wc, bc)
    return logits, emb


# speedup vs baseline: 2.6944x; 1.0254x over previous
import functools

import jax
import jax.numpy as jnp
from jax import lax
from jax.experimental import pallas as pl
from jax.experimental.pallas import tpu as pltpu
from jax.experimental.pallas import tpu_sc as plsc

N = 10000
IN = 128
H = 64
C = 32

NPAD = 10240
CH = 128
NC = 2
NS = 16
NW = NC * NS
ROWS_PER_TILE = NPAD // NS
NB = 5
NPOOL = 2 * NB


HH = H // NC


def _make_sc_agg(nchunks: int, with_deg: bool):
    out_type = [jax.ShapeDtypeStruct((NPAD, 128), jnp.float32)]
    if with_deg:
        out_type.append(jax.ShapeDtypeStruct((NC, NPAD), jnp.float32))

    scratch = [
        pltpu.VMEM((nchunks, CH), jnp.int32),
        pltpu.VMEM((nchunks, CH), jnp.int32),
        pltpu.VMEM((NPOOL, CH, HH), jnp.float32),
        pltpu.VMEM((CH,), jnp.float32),
        pltpu.VMEM_SHARED((NPAD, HH), jnp.float32),
        pltpu.VMEM_SHARED((NPAD,), jnp.float32),
        pltpu.VMEM_SHARED((N, HH), jnp.float32),
    ] + [pltpu.SemaphoreType.DMA] * (2 * NPOOL)

    mesh = plsc.VectorSubcoreMesh(core_axis_name="c", subcore_axis_name="s",
                                  num_cores=NC, num_subcores=NS)
    nch2 = nchunks // 2

    def body(table, src, dst, zh, z1, *rest):
        if with_deg:
            out, degout, sidx, didx, rows, ones, acc, dacc, tab = rest[:9]
        else:
            out, sidx, didx, rows, ones, acc, dacc, tab = rest[:8]
            degout = None
        sems = rest[-2 * NPOOL:]
        c = lax.axis_index("c")
        s = lax.axis_index("s")

        for i in range(CH // 16):
            ones[pl.ds(i * 16, 16)] = jnp.ones((16,), jnp.float32)

        r0 = s * ROWS_PER_TILE
        pltpu.sync_copy(zh.at[pl.ds(r0, ROWS_PER_TILE)],
                        acc.at[pl.ds(r0, ROWS_PER_TILE)])
        pltpu.sync_copy(z1.at[pl.ds(r0, ROWS_PER_TILE)],
                        dacc.at[pl.ds(r0, ROWS_PER_TILE)])
        tr = N // NS

        @pl.when(c == 0)
        def _():
            pltpu.sync_copy(table.at[pl.ds(s * tr, tr), pl.ds(0, HH)],
                            tab.at[pl.ds(s * tr, tr)])

        @pl.when(c == 1)
        def _():
            pltpu.sync_copy(table.at[pl.ds(s * tr, tr), pl.ds(HH, HH)],
                            tab.at[pl.ds(s * tr, tr)])

        pltpu.sync_copy(src.at[pl.ds(s * nchunks, nchunks)], sidx)
        pltpu.sync_copy(dst.at[pl.ds(s * nchunks, nchunks)], didx)
        plsc.subcore_barrier()

        def sidx_of(j):
            return sidx.at[j]

        def didx_of(j):
            return didx.at[j]

        for b in range(NB):
            pltpu.async_copy(tab.at[sidx_of(b)], rows.at[b], sems[b])

        def pair(p, carry):
            for u in range(NPOOL):
                j = p * NPOOL + u
                b = u
                bn = (u + NB) % NPOOL
                pltpu.make_async_copy(tab.at[sidx_of(j)], rows.at[b],
                                      sems[b]).wait()
                pltpu.async_copy(rows.at[b], acc.at[didx_of(j)],
                                 sems[NPOOL + b], add=True)
                if with_deg:
                    @pl.when((j < nch2) == (c == 0))
                    def _():
                        pltpu.sync_copy(ones, dacc.at[didx_of(j)], add=True)

                @pl.when(jnp.logical_and(j + NB < nchunks, j >= NB))
                def _():
                    pltpu.make_async_copy(rows.at[bn], acc.at[didx_of(0)],
                                          sems[NPOOL + bn]).wait()

                @pl.when(j + NB < nchunks)
                def _():
                    pltpu.async_copy(tab.at[sidx_of(j + NB)], rows.at[bn],
                                     sems[bn])
            return carry

        lax.fori_loop(0, nchunks // NPOOL, pair, 0)

        for b in range(NPOOL):
            pltpu.make_async_copy(rows.at[b], acc.at[didx_of(0)],
                                  sems[NPOOL + b]).wait()
        plsc.subcore_barrier()

        @pl.when(c == 0)
        def _():
            pltpu.sync_copy(acc.at[pl.ds(r0, ROWS_PER_TILE)],
                            out.at[pl.ds(r0, ROWS_PER_TILE), pl.ds(0, HH)])

        @pl.when(c == 1)
        def _():
            pltpu.sync_copy(acc.at[pl.ds(r0, ROWS_PER_TILE)],
                            out.at[pl.ds(r0, ROWS_PER_TILE), pl.ds(HH, HH)])
        if with_deg:
            pltpu.sync_copy(dacc.at[pl.ds(r0, ROWS_PER_TILE)],
                            degout.at[c, pl.ds(r0, ROWS_PER_TILE)])

    return pl.kernel(body, out_type=out_type, mesh=mesh, scratch_types=scratch,
                     compiler_params=pltpu.CompilerParams(use_tc_tiling_on_sc=False))


def _tc1_body(x_ref, wl_ref, wr_ref, a_ref):
    x = x_ref[...]
    y = jnp.dot(x, wl_ref[...], preferred_element_type=jnp.float32)
    r = jnp.dot(x, wr_ref[...], preferred_element_type=jnp.float32)
    a_ref[...] = jnp.concatenate([y, r], axis=-1)


def _tc2_body(p_ref, dpt_ref, a1_ref, b1_ref, wl_ref, wr_ref, a_ref):
    agg = p_ref[:N, :H]
    deg = dpt_ref[:N, 0] + dpt_ref[:N, 1]
    inv = 1.0 / jnp.maximum(deg, 1.0)
    r1 = a1_ref[:, H:]
    h = jnp.maximum(agg * inv[:, None] + r1 + b1_ref[...][None, :], 0.0)
    y = jnp.dot(h, wl_ref[...], preferred_element_type=jnp.float32)
    r = jnp.dot(h, wr_ref[...], preferred_element_type=jnp.float32)
    a_ref[...] = jnp.concatenate([y, r], axis=-1)


def _tc3_body(p_ref, dpt_ref, a2_ref, b2_ref, wc_ref, bc_ref, log_ref, emb_ref):
    agg = p_ref[:N, :H]
    deg = dpt_ref[:N, 0] + dpt_ref[:N, 1]
    inv = 1.0 / jnp.maximum(deg, 1.0)
    emb = agg * inv[:, None] + a2_ref[:, H:] + b2_ref[...][None, :]
    emb_ref[...] = emb
    log_ref[...] = (jnp.dot(emb, wc_ref[...], preferred_element_type=jnp.float32)
                    + bc_ref[...][None, :])


def _whole(shape):
    return pl.BlockSpec(shape, lambda i: tuple(0 for _ in shape))


_tc1 = pl.pallas_call(
    _tc1_body,
    grid=(1,),
    in_specs=[_whole((N, IN)), _whole((IN, H)), _whole((IN, H))],
    out_specs=[_whole((N, 2 * H))],
    out_shape=[jax.ShapeDtypeStruct((N, 2 * H), jnp.float32)],
)

_tc2 = pl.pallas_call(
    _tc2_body,
    grid=(1,),
    in_specs=[
        _whole((NPAD, 128)),
        _whole((NPAD, NC)),
        _whole((N, 2 * H)),
        _whole((H,)),
        _whole((H, H)),
        _whole((H, H)),
    ],
    out_specs=[_whole((N, 2 * H))],
    out_shape=[jax.ShapeDtypeStruct((N, 2 * H), jnp.float32)],
)

_tc3 = pl.pallas_call(
    _tc3_body,
    grid=(1,),
    in_specs=[
        _whole((NPAD, 128)),
        _whole((NPAD, NC)),
        _whole((N, 2 * H)),
        _whole((H,)),
        _whole((H, C)),
        _whole((C,)),
    ],
    out_specs=[_whole((N, C)), _whole((N, H))],
    out_shape=[jax.ShapeDtypeStruct((N, C), jnp.float32),
               jax.ShapeDtypeStruct((N, H), jnp.float32)],
)


def _make_edge_prep(E: int, e_pad: int):
    nrows = e_pad // CH

    def body(ei_ref, srcm_ref, dstm_ref):
        sm = ei_ref[0].reshape(nrows, CH)
        dm = ei_ref[1].reshape(nrows, CH)
        flat = (lax.broadcasted_iota(jnp.int32, (nrows, CH), 0) * CH +
                lax.broadcasted_iota(jnp.int32, (nrows, CH), 1))
        mask = flat < E
        srcm_ref[...] = jnp.where(mask, sm, 0)
        dstm_ref[...] = jnp.where(mask, dm, NPAD - 1)

    return pl.pallas_call(
        body,
        grid=(1,),
        in_specs=[pl.BlockSpec((2, e_pad), lambda i: (0, 0))],
        out_specs=[_whole((nrows, CH)), _whole((nrows, CH))],
        out_shape=[jax.ShapeDtypeStruct((nrows, CH), jnp.int32)] * 2,
    )


def kernel(x, edge_index, W1l, b1, W1r, W2l, b2, W2r, Wc, bc):
    E = edge_index.shape[1]
    nchunks = -(-(-(-E // (NS * CH))) // NPOOL) * NPOOL
    e_pad = NS * nchunks * CH

    srcm, dstm = _make_edge_prep(E, e_pad)(edge_index)

    zh = jnp.zeros((NPAD, HH), jnp.float32)
    z1 = jnp.zeros((NPAD,), jnp.float32)

    sc_agg1 = _make_sc_agg(nchunks, with_deg=True)
    sc_agg2 = _make_sc_agg(nchunks, with_deg=False)

    (a1,) = _tc1(x, W1l, W1r)
    p1, dp = sc_agg1(a1, srcm, dstm, zh, z1)
    dpt = dp.T
    (a2,) = _tc2(p1, dpt, a1, b1, W2l, W2r)
    p2 = sc_agg2(a2, srcm, dstm, zh, z1)
    p2 = p2[0] if isinstance(p2, (list, tuple)) else p2
    logits, emb = _tc3(p2, dpt, a2, b2, Wc, bc)
    return logits, emb


# speedup vs baseline: 2.6973x; 1.0011x over previous
import jax
import jax.numpy as jnp
from jax import lax
from jax.experimental import pallas as pl
from jax.experimental.pallas import tpu as pltpu
from jax.experimental.pallas import tpu_sc as plsc

N = 10000
IN = 128
H = 64
C = 32

NPAD = 10240
CH = 128
NC = 2
NS = 16
NW = NC * NS
ROWS_PER_TILE = NPAD // NS
NB = 5
NPOOL = 2 * NB


HH = H // NC


def _make_sc_agg(nchunks: int, with_deg: bool):
    out_type = [jax.ShapeDtypeStruct((NPAD, 128), jnp.float32)]
    if with_deg:
        out_type.append(jax.ShapeDtypeStruct((NC, NPAD), jnp.float32))

    scratch = [
        pltpu.VMEM((nchunks, CH), jnp.int32),
        pltpu.VMEM((nchunks, CH), jnp.int32),
        pltpu.VMEM((NPOOL, CH, HH), jnp.float32),
        pltpu.VMEM((CH,), jnp.float32),
        pltpu.VMEM_SHARED((NPAD, HH), jnp.float32),
        pltpu.VMEM_SHARED((NPAD,), jnp.float32),
        pltpu.VMEM_SHARED((N, HH), jnp.float32),
    ] + [pltpu.SemaphoreType.DMA] * (2 * NPOOL)

    mesh = plsc.VectorSubcoreMesh(core_axis_name="c", subcore_axis_name="s",
                                  num_cores=NC, num_subcores=NS)
    nch2 = nchunks // 2

    def body(table, src, dst, zh, z1, *rest):
        if with_deg:
            out, degout, sidx, didx, rows, ones, acc, dacc, tab = rest[:9]
        else:
            out, sidx, didx, rows, ones, acc, dacc, tab = rest[:8]
            degout = None
        sems = rest[-2 * NPOOL:]
        c = lax.axis_index("c")
        s = lax.axis_index("s")

        for i in range(CH // 16):
            ones[pl.ds(i * 16, 16)] = jnp.ones((16,), jnp.float32)

        r0 = s * ROWS_PER_TILE
        pltpu.sync_copy(zh.at[pl.ds(r0, ROWS_PER_TILE)],
                        acc.at[pl.ds(r0, ROWS_PER_TILE)])
        pltpu.sync_copy(z1.at[pl.ds(r0, ROWS_PER_TILE)],
                        dacc.at[pl.ds(r0, ROWS_PER_TILE)])
        tr = N // NS

        @pl.when(c == 0)
        def _():
            pltpu.sync_copy(table.at[pl.ds(s * tr, tr), pl.ds(0, HH)],
                            tab.at[pl.ds(s * tr, tr)])

        @pl.when(c == 1)
        def _():
            pltpu.sync_copy(table.at[pl.ds(s * tr, tr), pl.ds(HH, HH)],
                            tab.at[pl.ds(s * tr, tr)])

        pltpu.sync_copy(src.at[pl.ds(s * nchunks, nchunks)], sidx)
        pltpu.sync_copy(dst.at[pl.ds(s * nchunks, nchunks)], didx)
        plsc.subcore_barrier()

        def sidx_of(j):
            return sidx.at[j]

        def didx_of(j):
            return didx.at[j]

        for b in range(NB):
            pltpu.async_copy(tab.at[sidx_of(b)], rows.at[b], sems[b])

        def pair(p, carry):
            for u in range(NPOOL):
                j = p * NPOOL + u
                b = u
                bn = (u + NB) % NPOOL
                pltpu.make_async_copy(tab.at[sidx_of(j)], rows.at[b],
                                      sems[b]).wait()
                pltpu.async_copy(rows.at[b], acc.at[didx_of(j)],
                                 sems[NPOOL + b], add=True)
                if with_deg:
                    @pl.when((j < nch2) == (c == 0))
                    def _():
                        pltpu.sync_copy(ones, dacc.at[didx_of(j)], add=True)

                @pl.when(jnp.logical_and(j + NB < nchunks, j >= NB))
                def _():
                    pltpu.make_async_copy(rows.at[bn], acc.at[didx_of(0)],
                                          sems[NPOOL + bn]).wait()

                @pl.when(j + NB < nchunks)
                def _():
                    pltpu.async_copy(tab.at[sidx_of(j + NB)], rows.at[bn],
                                     sems[bn])
            return carry

        lax.fori_loop(0, nchunks // NPOOL, pair, 0)

        for b in range(NPOOL):
            pltpu.make_async_copy(rows.at[b], acc.at[didx_of(0)],
                                  sems[NPOOL + b]).wait()
        plsc.subcore_barrier()

        @pl.when(c == 0)
        def _():
            pltpu.sync_copy(acc.at[pl.ds(r0, ROWS_PER_TILE)],
                            out.at[pl.ds(r0, ROWS_PER_TILE), pl.ds(0, HH)])

        @pl.when(c == 1)
        def _():
            pltpu.sync_copy(acc.at[pl.ds(r0, ROWS_PER_TILE)],
                            out.at[pl.ds(r0, ROWS_PER_TILE), pl.ds(HH, HH)])
        if with_deg:
            pltpu.sync_copy(dacc.at[pl.ds(r0, ROWS_PER_TILE)],
                            degout.at[c, pl.ds(r0, ROWS_PER_TILE)])

    return pl.kernel(body, out_type=out_type, mesh=mesh, scratch_types=scratch,
                     compiler_params=pltpu.CompilerParams(use_tc_tiling_on_sc=False))


def _tc1_body(x_ref, wl_ref, wr_ref, a_ref):
    x = x_ref[...]
    y = jnp.dot(x, wl_ref[...], preferred_element_type=jnp.float32)
    r = jnp.dot(x, wr_ref[...], preferred_element_type=jnp.float32)
    a_ref[...] = jnp.concatenate([y, r], axis=-1)


def _tc2_body(p_ref, dpt_ref, a1_ref, b1_ref, wl_ref, wr_ref, a_ref):
    agg = p_ref[:N, :H]
    deg = dpt_ref[:N, 0] + dpt_ref[:N, 1]
    inv = 1.0 / jnp.maximum(deg, 1.0)
    r1 = a1_ref[:, H:]
    h = jnp.maximum(agg * inv[:, None] + r1 + b1_ref[...][None, :], 0.0)
    y = jnp.dot(h, wl_ref[...], preferred_element_type=jnp.float32)
    r = jnp.dot(h, wr_ref[...], preferred_element_type=jnp.float32)
    a_ref[...] = jnp.concatenate([y, r], axis=-1)


def _tc3_body(p_ref, dpt_ref, a2_ref, b2_ref, wc_ref, bc_ref, log_ref, emb_ref):
    agg = p_ref[:N, :H]
    deg = dpt_ref[:N, 0] + dpt_ref[:N, 1]
    inv = 1.0 / jnp.maximum(deg, 1.0)
    emb = agg * inv[:, None] + a2_ref[:, H:] + b2_ref[...][None, :]
    emb_ref[...] = emb
    log_ref[...] = (jnp.dot(emb, wc_ref[...], preferred_element_type=jnp.float32)
                    + bc_ref[...][None, :])


def _whole(shape):
    return pl.BlockSpec(shape, lambda i: tuple(0 for _ in shape))


_tc1 = pl.pallas_call(
    _tc1_body,
    grid=(1,),
    in_specs=[_whole((N, IN)), _whole((IN, H)), _whole((IN, H))],
    out_specs=[_whole((N, 2 * H))],
    out_shape=[jax.ShapeDtypeStruct((N, 2 * H), jnp.float32)],
)

_tc2 = pl.pallas_call(
    _tc2_body,
    grid=(1,),
    in_specs=[
        _whole((NPAD, 128)),
        _whole((NPAD, NC)),
        _whole((N, 2 * H)),
        _whole((H,)),
        _whole((H, H)),
        _whole((H, H)),
    ],
    out_specs=[_whole((N, 2 * H))],
    out_shape=[jax.ShapeDtypeStruct((N, 2 * H), jnp.float32)],
)

_tc3 = pl.pallas_call(
    _tc3_body,
    grid=(1,),
    in_specs=[
        _whole((NPAD, 128)),
        _whole((NPAD, NC)),
        _whole((N, 2 * H)),
        _whole((H,)),
        _whole((H, C)),
        _whole((C,)),
    ],
    out_specs=[_whole((N, C)), _whole((N, H))],
    out_shape=[jax.ShapeDtypeStruct((N, C), jnp.float32),
               jax.ShapeDtypeStruct((N, H), jnp.float32)],
)


def _make_edge_prep(E: int, e_pad: int):
    nrows = e_pad // CH

    def body(ei_ref, srcm_ref, dstm_ref):
        sm = ei_ref[0].reshape(nrows, CH)
        dm = ei_ref[1].reshape(nrows, CH)
        flat = (lax.broadcasted_iota(jnp.int32, (nrows, CH), 0) * CH +
                lax.broadcasted_iota(jnp.int32, (nrows, CH), 1))
        mask = flat < E
        srcm_ref[...] = jnp.where(mask, sm, 0)
        dstm_ref[...] = jnp.where(mask, dm, NPAD - 1)

    return pl.pallas_call(
        body,
        grid=(1,),
        in_specs=[pl.BlockSpec((2, e_pad), lambda i: (0, 0))],
        out_specs=[_whole((nrows, CH)), _whole((nrows, CH))],
        out_shape=[jax.ShapeDtypeStruct((nrows, CH), jnp.int32)] * 2,
    )


def kernel(x, edge_index, W1l, b1, W1r, W2l, b2, W2r, Wc, bc):
    E = edge_index.shape[1]
    nchunks = -(-(-(-E // (NS * CH))) // NPOOL) * NPOOL
    e_pad = NS * nchunks * CH

    srcm, dstm = _make_edge_prep(E, e_pad)(edge_index)

    zh = jnp.zeros((NPAD, HH), jnp.float32)
    z1 = jnp.zeros((NPAD,), jnp.float32)

    sc_agg1 = _make_sc_agg(nchunks, with_deg=True)
    sc_agg2 = _make_sc_agg(nchunks, with_deg=False)

    (a1,) = _tc1(x, W1l, W1r)
    p1, dp = sc_agg1(a1, srcm, dstm, zh, z1)
    dpt = dp.T
    (a2,) = _tc2(p1, dpt, a1, b1, W2l, W2r)
    p2 = sc_agg2(a2, srcm, dstm, zh, z1)
    p2 = p2[0] if isinstance(p2, (list, tuple)) else p2
    logits, emb = _tc3(p2, dpt, a2, b2, Wc, bc)
    return logits, emb
